# Initial kernel scaffold; baseline (speedup 1.0000x reference)
import functools

import jax
import jax.numpy as jnp
from jax.experimental import pallas as pl

_N_SPECIES = 50000
_N_SPATIAL = 10000
_HID = 32
_OUT = 16
_EPS = 1e-6


def _edge_softmax_pass(a_src, a_dst, qattr, src, dst, h_src, num_dst):
    alpha = a_src[src] + a_dst[dst] + qattr
    alpha = jnp.where(alpha > 0, alpha, 0.2 * alpha)
    ex = jnp.exp(alpha)
    denom = jax.ops.segment_sum(ex, dst, num_segments=num_dst)
    num = jax.ops.segment_sum(h_src[src] * ex[..., None], dst,
                              num_segments=num_dst)
    return num, denom


def _gat_layer(x_src, x_dst, edge_index, edge_attr, p, num_dst, heads,
               out_ch, concat, self_loops):
    src = edge_index[0]
    dst = edge_index[1]
    attr = edge_attr[:, 0]
    h_src = (x_src @ p['W_src']).reshape(-1, heads, out_ch)
    h_dst = (x_dst @ p['W_dst']).reshape(-1, heads, out_ch)
    a_src = (h_src * p['att_src'][None]).sum(-1)
    a_dst = (h_dst * p['att_dst'][None]).sum(-1)
    q = (p['W_edge'].reshape(heads, out_ch) * p['att_edge']).sum(-1)
    qattr = attr[:, None] * q[None, :]
    num, denom = _edge_softmax_pass(a_src, a_dst, qattr, src, dst, h_src,
                                    num_dst)
    if self_loops:
        ones = jnp.ones_like(attr)
        deg = jax.ops.segment_sum(ones, dst, num_segments=num_dst)
        asum = jax.ops.segment_sum(attr, dst, num_segments=num_dst)
        loop_attr = asum / jnp.maximum(deg, 1.0)
        alpha_s = a_src + a_dst + loop_attr[:, None] * q[None, :]
        alpha_s = jnp.where(alpha_s > 0, alpha_s, 0.2 * alpha_s)
        ex_s = jnp.exp(alpha_s)
        num = num + h_src * ex_s[..., None]
        denom = denom + ex_s
    out = num / jnp.maximum(denom, 1e-16)[..., None]
    if concat:
        out = out.reshape(num_dst, heads * out_ch)
    else:
        out = out.mean(axis=1)
    return out + p['bias'][None]


def _fc_kernel(x_ref, w_ref, b_ref, o_ref):
    o_ref[...] = (
        jnp.dot(x_ref[...], w_ref[...], preferred_element_type=jnp.float32)
        + b_ref[...]
    )


def _fc(x, w, b):
    n, k = x.shape
    m = w.shape[1]
    blk = 2000
    return pl.pallas_call(
        _fc_kernel,
        out_shape=jax.ShapeDtypeStruct((n, m), jnp.float32),
        grid=(n // blk,),
        in_specs=[
            pl.BlockSpec((blk, k), lambda i: (i, 0)),
            pl.BlockSpec((k, m), lambda i: (0, 0)),
            pl.BlockSpec((m,), lambda i: (0,)),
        ],
        out_specs=pl.BlockSpec((blk, m), lambda i: (i, 0)),
    )(x, w, b)


def kernel(species_x_mean, species_x_std, traits_nanmask, species_x_gen,
           species_x_phylo, spatial_x, spatial_global_data,
           spatial_spatial_edge_index, spatial_spatial_edge_attr,
           spatial_species_edge_index, spatial_species_edge_attr,
           species_species_edge_index, species_species_edge_attr, params):
    p = params
    vis = (~traits_nanmask).astype(jnp.float32)
    sp_in = jnp.concatenate(
        [species_x_mean, species_x_std, vis, species_x_gen, species_x_phylo],
        axis=1)
    sp_in = jax.nn.relu(sp_in @ p['W_lin'] + p['b_lin'])

    space_in = jnp.concatenate([spatial_x, spatial_global_data], axis=1)
    h = _gat_layer(space_in, space_in, spatial_spatial_edge_index,
                   spatial_spatial_edge_attr, p['space_g1'], _N_SPATIAL, 4,
                   _HID, False, True)
    h = jax.nn.relu(h)
    space_emb = _gat_layer(h, h, spatial_spatial_edge_index,
                           spatial_spatial_edge_attr, p['space_g2'],
                           _N_SPATIAL, 4, _HID, False, True)
    space_emb = jax.nn.relu(space_emb)

    species_part = jnp.concatenate([species_x_gen, species_x_phylo], axis=1)
    s2s = _gat_layer(space_emb, species_part, spatial_species_edge_index,
                     spatial_species_edge_attr, p['bip'], _N_SPECIES, 1,
                     _HID, True, False)
    s2s = jax.nn.relu(s2s)

    sp_in = jnp.concatenate([s2s, sp_in], axis=1)
    h = _gat_layer(sp_in, sp_in, species_species_edge_index,
                   species_species_edge_attr, p['spec_g1'], _N_SPECIES, 4,
                   _HID, False, True)
    h = jax.nn.relu(h)
    emb = _gat_layer(h, h, species_species_edge_index,
                     species_species_edge_attr, p['spec_g2'], _N_SPECIES, 4,
                     _HID, False, True)

    out = _fc(emb, p['W_fc'], p['b_fc'])
    pred_mean = out[:, :_OUT]
    pred_std = jax.nn.softplus(out[:, _OUT:]) + _EPS
    return (pred_mean, pred_std)


# speedup vs baseline: 1.1659x; 1.1659x over previous
import functools

import jax
import jax.numpy as jnp
from jax.experimental import pallas as pl

_N_SPECIES = 50000
_N_SPATIAL = 10000
_HID = 32
_OUT = 16
_EPS = 1e-6


def _edge_softmax_pass(a_src, a_dst, qattr, src, dst, h_src, num_dst):
    alpha = a_src[src] + a_dst[dst] + qattr
    alpha = jnp.where(alpha > 0, alpha, 0.2 * alpha)
    ex = jnp.exp(alpha)
    denom = jax.ops.segment_sum(ex, dst, num_segments=num_dst)
    num = jax.ops.segment_sum(h_src[src] * ex[..., None], dst,
                              num_segments=num_dst)
    return num, denom


def _gat_layer(x_src, x_dst, edge_index, edge_attr, p, num_dst, heads,
               out_ch, concat, self_loops):
    src = edge_index[0]
    dst = edge_index[1]
    attr = edge_attr[:, 0]
    h_src = (x_src @ p['W_src']).reshape(-1, heads, out_ch)
    h_dst = (x_dst @ p['W_dst']).reshape(-1, heads, out_ch)
    a_src = (h_src * p['att_src'][None]).sum(-1)
    a_dst = (h_dst * p['att_dst'][None]).sum(-1)
    q = (p['W_edge'].reshape(heads, out_ch) * p['att_edge']).sum(-1)
    qattr = attr[:, None] * q[None, :]
    num, denom = _edge_softmax_pass(a_src, a_dst, qattr, src, dst, h_src,
                                    num_dst)
    if self_loops:
        ones = jnp.ones_like(attr)
        deg = jax.ops.segment_sum(ones, dst, num_segments=num_dst)
        asum = jax.ops.segment_sum(attr, dst, num_segments=num_dst)
        loop_attr = asum / jnp.maximum(deg, 1.0)
        alpha_s = a_src + a_dst + loop_attr[:, None] * q[None, :]
        alpha_s = jnp.where(alpha_s > 0, alpha_s, 0.2 * alpha_s)
        ex_s = jnp.exp(alpha_s)
        num = num + h_src * ex_s[..., None]
        denom = denom + ex_s
    out = num / jnp.maximum(denom, 1e-16)[..., None]
    if concat:
        out = out.reshape(num_dst, heads * out_ch)
    else:
        out = out.mean(axis=1)
    return out + p['bias'][None]


def _fc_kernel(x_ref, w_ref, b_ref, o_ref):
    o_ref[...] = (
        jnp.dot(x_ref[...], w_ref[...], preferred_element_type=jnp.float32)
        + b_ref[...]
    )


def _fc(x, w, b):
    n, k = x.shape
    m = w.shape[1]
    blk = 1000
    return pl.pallas_call(
        _fc_kernel,
        out_shape=jax.ShapeDtypeStruct((n, m), jnp.float32),
        grid=(n // blk,),
        in_specs=[
            pl.BlockSpec((blk, k), lambda i: (i, 0)),
            pl.BlockSpec((k, m), lambda i: (0, 0)),
            pl.BlockSpec((1, m), lambda i: (0, 0)),
        ],
        out_specs=pl.BlockSpec((blk, m), lambda i: (i, 0)),
    )(x, w, b.reshape(1, m))


def kernel(species_x_mean, species_x_std, traits_nanmask, species_x_gen,
           species_x_phylo, spatial_x, spatial_global_data,
           spatial_spatial_edge_index, spatial_spatial_edge_attr,
           spatial_species_edge_index, spatial_species_edge_attr,
           species_species_edge_index, species_species_edge_attr, params):
    p = params
    vis = (~traits_nanmask).astype(jnp.float32)
    sp_in = jnp.concatenate(
        [species_x_mean, species_x_std, vis, species_x_gen, species_x_phylo],
        axis=1)
    sp_in = jax.nn.relu(sp_in @ p['W_lin'] + p['b_lin'])

    space_in = jnp.concatenate([spatial_x, spatial_global_data], axis=1)
    h = _gat_layer(space_in, space_in, spatial_spatial_edge_index,
                   spatial_spatial_edge_attr, p['space_g1'], _N_SPATIAL, 4,
                   _HID, False, True)
    h = jax.nn.relu(h)
    space_emb = _gat_layer(h, h, spatial_spatial_edge_index,
                           spatial_spatial_edge_attr, p['space_g2'],
                           _N_SPATIAL, 4, _HID, False, True)
    space_emb = jax.nn.relu(space_emb)

    species_part = jnp.concatenate([species_x_gen, species_x_phylo], axis=1)
    s2s = _gat_layer(space_emb, species_part, spatial_species_edge_index,
                     spatial_species_edge_attr, p['bip'], _N_SPECIES, 1,
                     _HID, True, False)
    s2s = jax.nn.relu(s2s)

    sp_in = jnp.concatenate([s2s, sp_in], axis=1)
    h = _gat_layer(sp_in, sp_in, species_species_edge_index,
                   species_species_edge_attr, p['spec_g1'], _N_SPECIES, 4,
                   _HID, False, True)
    h = jax.nn.relu(h)
    emb = _gat_layer(h, h, species_species_edge_index,
                     species_species_edge_attr, p['spec_g2'], _N_SPECIES, 4,
                     _HID, False, True)

    out = _fc(emb, p['W_fc'], p['b_fc'])
    pred_mean = out[:, :_OUT]
    pred_std = jax.nn.softplus(out[:, _OUT:]) + _EPS
    return (pred_mean, pred_std)


# speedup vs baseline: 7.2417x; 6.2112x over previous
import functools

import jax
import jax.numpy as jnp
from jax import lax
from jax.experimental import pallas as pl
from jax.experimental.pallas import tpu as pltpu
from jax.experimental.pallas import tpu_sc as plsc

_N_SPECIES = 50000
_N_SPATIAL = 10000
_HID = 32
_OUT = 16
_EPS = 1e-6

_G = 128
_ZR = 32
_NT = 16
_NC = 2


def _rup(x, m):
    return ((x + m - 1) // m) * m


def _make_edge_pass(n_src, n_dst, e_pad, heads, n_halves, with_aux):
    na_s = _rup(n_src + 1, 8)
    na_d = _rup(n_dst + 1, 8)
    hr = _rup((n_dst + n_halves - 1) // n_halves, 8)
    nz = _rup(hr + 1, _NT * _ZR)
    split_edges = heads == 1
    hpc = heads // _NC if not split_edges else 1
    e_scan = e_pad // _NC if split_edges else e_pad
    ept = e_scan // _NT
    n_chunks = ept // _G
    trows = nz // _NT
    n_zchunks = trows // _ZR

    mesh = plsc.VectorSubcoreMesh(core_axis_name="c", subcore_axis_name="s")

    out_type = [
        jax.ShapeDtypeStruct((_NC, hpc, n_halves, nz, _HID), jnp.float32),
        jax.ShapeDtypeStruct((_NC, hpc, n_halves, nz), jnp.float32),
    ]
    if with_aux:
        out_type.append(
            jax.ShapeDtypeStruct((_NC, 2, n_halves, nz), jnp.float32))

    scratch_types = [
        pltpu.VMEM((na_s,), jnp.float32),
        pltpu.VMEM((na_d,), jnp.float32),
        pltpu.VMEM((_G,), jnp.int32),
        pltpu.VMEM((_G,), jnp.int32),
        pltpu.VMEM((_G,), jnp.int32),
        pltpu.VMEM((_G,), jnp.float32),
        pltpu.VMEM((_G,), jnp.float32),
        pltpu.VMEM((_G,), jnp.float32),
        pltpu.VMEM((_G, _HID), jnp.float32),
        pltpu.VMEM((_ZR, _HID), jnp.float32),
        pltpu.VMEM((_ZR,), jnp.float32),
        pltpu.VMEM((_ZR, _HID), jnp.float32),
        pltpu.VMEM((_ZR,), jnp.float32),
        pltpu.VMEM((16,), jnp.float32),
        pltpu.SemaphoreType.DMA,
        pltpu.VMEM_SHARED((nz, _HID), jnp.float32),
        pltpu.VMEM_SHARED((nz,), jnp.float32),
    ]
    if with_aux:
        scratch_types += [
            pltpu.VMEM_SHARED((nz,), jnp.float32),
            pltpu.VMEM_SHARED((nz,), jnp.float32),
        ]

    @functools.partial(
        pl.kernel, mesh=mesh, out_type=out_type,
        scratch_types=scratch_types,
        compiler_params=pltpu.CompilerParams(needs_layout_passes=False,
                                             use_tc_tiling_on_sc=False))
    def edge_pass(*refs):
        (src_h, dst_h, attr_h, asrc_h, adst_h, hsrc_h, q_h) = refs[:7]
        if with_aux:
            (num_h, den_h, aux_h) = refs[7:10]
            scr = refs[10:]
        else:
            (num_h, den_h) = refs[7:9]
            aux_h = None
            scr = refs[9:]
        if with_aux:
            (a_src_v, a_dst_v, si, di, dl, at, exb, ones, rows,
             zrows, zvec, strows, stvec, qb, sem,
             num_sh, den_sh, deg_sh, asum_sh) = scr
        else:
            (a_src_v, a_dst_v, si, di, dl, at, exb, ones, rows,
             zrows, zvec, strows, stvec, qb, sem,
             num_sh, den_sh) = scr
            deg_sh = asum_sh = None

        c = lax.axis_index("c")
        s = lax.axis_index("s")
        ebase0 = (c * e_scan if split_edges else 0) + s * ept
        r0 = s * trows

        zero16 = jnp.zeros((16,), jnp.float32)
        one16 = jnp.ones((16,), jnp.float32)

        def init_zrows(r, _):
            zrows[r, pl.ds(0, 16)] = zero16
            zrows[r, pl.ds(16, 16)] = zero16
            return 0
        lax.fori_loop(0, _ZR, init_zrows, 0)

        def init_vecs(j, _):
            zvec[pl.ds(j * 16, 16)] = zero16
            return 0
        lax.fori_loop(0, _ZR // 16, init_vecs, 0)

        def init_ones(j, _):
            ones[pl.ds(j * 16, 16)] = one16
            return 0
        lax.fori_loop(0, _G // 16, init_ones, 0)

        for hi in range(hpc):
            hd = c * hpc + hi if not split_edges else 0
            pltpu.sync_copy(asrc_h.at[hd], a_src_v)
            pltpu.sync_copy(adst_h.at[hd], a_dst_v)
            pltpu.sync_copy(q_h.at[hd], qb)

            for half in range(n_halves):
                lo = half * hr
                span = min(hr, n_dst - lo)
                do_aux = with_aux and hi == 0

                def zero_body(z, _):
                    r = r0 + z * _ZR
                    pltpu.sync_copy(zrows, num_sh.at[pl.ds(r, _ZR)])
                    pltpu.sync_copy(zvec, den_sh.at[pl.ds(r, _ZR)])
                    if do_aux:
                        pltpu.sync_copy(zvec, deg_sh.at[pl.ds(r, _ZR)])
                        pltpu.sync_copy(zvec, asum_sh.at[pl.ds(r, _ZR)])
                    return 0
                lax.fori_loop(0, n_zchunks, zero_body, 0)
                plsc.subcore_barrier()

                def chunk_body(k, _):
                    eb = ebase0 + k * _G
                    pltpu.sync_copy(src_h.at[pl.ds(eb, _G)], si)
                    pltpu.sync_copy(dst_h.at[pl.ds(eb, _G)], di)
                    pltpu.sync_copy(attr_h.at[pl.ds(eb, _G)], at)
                    pltpu.async_copy(hsrc_h.at[hd].at[si], rows, sem).wait()

                    qv = qb[...]

                    def grp_body(g, _):
                        i0 = g * 16
                        sv = si[pl.ds(i0, 16)]
                        dv = di[pl.ds(i0, 16)]
                        av = at[pl.ds(i0, 16)]
                        a1 = plsc.load_gather(a_src_v, [sv])
                        a2 = plsc.load_gather(a_dst_v, [dv])
                        al = a1 + a2 + qv * av
                        al = jnp.where(al > 0, al, 0.2 * al)
                        ex = jnp.exp(al)
                        exb[pl.ds(i0, 16)] = ex
                        loc = dv - lo
                        owned = (loc >= 0) & (loc < span)
                        dl[pl.ds(i0, 16)] = jnp.where(
                            owned, loc, jnp.full((16,), span, jnp.int32))
                        for j in range(16):
                            r = i0 + j
                            bj = plsc.load_gather(
                                exb,
                                [jnp.broadcast_to(r, (16,)).astype(jnp.int32)])
                            rows[r, pl.ds(0, 16)] = rows[r, pl.ds(0, 16)] * bj
                            rows[r, pl.ds(16, 16)] = (
                                rows[r, pl.ds(16, 16)] * bj)
                        return 0
                    lax.fori_loop(0, _G // 16, grp_body, 0)

                    pltpu.sync_copy(rows, num_sh.at[dl], add=True)
                    pltpu.sync_copy(exb, den_sh.at[dl], add=True)
                    if do_aux:
                        pltpu.sync_copy(ones, deg_sh.at[dl], add=True)
                        pltpu.sync_copy(at, asum_sh.at[dl], add=True)
                    return 0
                lax.fori_loop(0, n_chunks, chunk_body, 0)
                plsc.subcore_barrier()

                def dump_body(z, _):
                    r = r0 + z * _ZR
                    pltpu.sync_copy(num_sh.at[pl.ds(r, _ZR)], strows)
                    pltpu.sync_copy(
                        strows, num_h.at[c, hi, half].at[pl.ds(r, _ZR)])
                    pltpu.sync_copy(den_sh.at[pl.ds(r, _ZR)], stvec)
                    pltpu.sync_copy(
                        stvec, den_h.at[c, hi, half].at[pl.ds(r, _ZR)])
                    if do_aux:
                        pltpu.sync_copy(deg_sh.at[pl.ds(r, _ZR)], stvec)
                        pltpu.sync_copy(
                            stvec, aux_h.at[c, 0, half].at[pl.ds(r, _ZR)])
                        pltpu.sync_copy(asum_sh.at[pl.ds(r, _ZR)], stvec)
                        pltpu.sync_copy(
                            stvec, aux_h.at[c, 1, half].at[pl.ds(r, _ZR)])
                    return 0
                lax.fori_loop(0, n_zchunks, dump_body, 0)
                plsc.subcore_barrier()

    return edge_pass, nz, na_s, na_d, hr, hpc


def _pad_edges(edge_index, edge_attr, n_dst):
    e = edge_index.shape[1]
    e_pad = _rup(e, _NC * _NT * _G)
    pad = e_pad - e
    src = jnp.concatenate([edge_index[0],
                           jnp.zeros((pad,), edge_index.dtype)])
    dst = jnp.concatenate([edge_index[1],
                           jnp.full((pad,), n_dst, edge_index.dtype)])
    attr = jnp.concatenate([edge_attr[:, 0],
                            jnp.zeros((pad,), edge_attr.dtype)])
    return src, dst, attr, e_pad


def _unhalve(x, n_dst, hr):
    n_halves = x.shape[0]
    parts = []
    for half in range(n_halves):
        span = min(hr, n_dst - half * hr)
        parts.append(x[half, :span])
    return jnp.concatenate(parts, axis=0)


def _gat_layer_sc(x_src, x_dst, src, dst, attr, e_pad, p, n_src, n_dst,
                  heads, n_halves, concat, self_loops, aux_in=None):
    with_aux = self_loops and aux_in is None
    edge_pass, nz, na_s, na_d, hr, hpc = _make_edge_pass(
        n_src, n_dst, e_pad, heads, n_halves, with_aux)

    h_src = (x_src @ p['W_src']).reshape(n_src, heads, _HID)
    h_dst = (x_dst @ p['W_dst']).reshape(n_dst, heads, _HID)
    a_src = (h_src * p['att_src'][None]).sum(-1)
    a_dst = (h_dst * p['att_dst'][None]).sum(-1)
    q = (p['W_edge'].reshape(heads, _HID) * p['att_edge']).sum(-1)

    asrc_t = jnp.zeros((heads, na_s), jnp.float32).at[:, :n_src].set(a_src.T)
    adst_t = jnp.zeros((heads, na_d), jnp.float32).at[:, :n_dst].set(a_dst.T)
    hsrc_t = jnp.concatenate(
        [h_src.transpose(1, 0, 2),
         jnp.zeros((heads, 1, _HID), jnp.float32)], axis=1)
    q_splat = jnp.broadcast_to(q[:, None], (heads, 16)).astype(jnp.float32)

    res = edge_pass(src, dst, attr, asrc_t, adst_t, hsrc_t, q_splat)
    if with_aux:
        num_o, den_o, aux_o = res
        deg = _unhalve(aux_o[0, 0], n_dst, hr)
        asum = _unhalve(aux_o[0, 1], n_dst, hr)
    else:
        num_o, den_o = res
        deg, asum = aux_in if aux_in is not None else (None, None)

    if heads == 1:
        num_o = num_o.sum(0, keepdims=True)
        den_o = den_o.sum(0, keepdims=True)
        num = _unhalve(num_o[0, 0], n_dst, hr)[None]
        den = _unhalve(den_o[0, 0], n_dst, hr)[None]
    else:
        num = jnp.stack([
            _unhalve(num_o[h // hpc, h % hpc], n_dst, hr)
            for h in range(heads)])
        den = jnp.stack([
            _unhalve(den_o[h // hpc, h % hpc], n_dst, hr)
            for h in range(heads)])

    if self_loops:
        loop_attr = asum / jnp.maximum(deg, 1.0)
        alpha_s = a_src + a_dst + loop_attr[:, None] * q[None, :]
        alpha_s = jnp.where(alpha_s > 0, alpha_s, 0.2 * alpha_s)
        ex_s = jnp.exp(alpha_s)
        num = num + h_src.transpose(1, 0, 2) * ex_s.T[:, :, None]
        den = den + ex_s.T
    out = num / jnp.maximum(den, 1e-16)[..., None]
    if concat:
        out = out.transpose(1, 0, 2).reshape(n_dst, heads * _HID)
    else:
        out = out.mean(axis=0)
    return out + p['bias'][None], (deg, asum)


def _fc_kernel(x_ref, w_ref, b_ref, o_ref):
    o_ref[...] = (
        jnp.dot(x_ref[...], w_ref[...], preferred_element_type=jnp.float32)
        + b_ref[...]
    )


def _fc(x, w, b):
    n, k = x.shape
    m = w.shape[1]
    blk = 1000
    return pl.pallas_call(
        _fc_kernel,
        out_shape=jax.ShapeDtypeStruct((n, m), jnp.float32),
        grid=(n // blk,),
        in_specs=[
            pl.BlockSpec((blk, k), lambda i: (i, 0)),
            pl.BlockSpec((k, m), lambda i: (0, 0)),
            pl.BlockSpec((1, m), lambda i: (0, 0)),
        ],
        out_specs=pl.BlockSpec((blk, m), lambda i: (i, 0)),
    )(x, w, b.reshape(1, m))


def kernel(species_x_mean, species_x_std, traits_nanmask, species_x_gen,
           species_x_phylo, spatial_x, spatial_global_data,
           spatial_spatial_edge_index, spatial_spatial_edge_attr,
           spatial_species_edge_index, spatial_species_edge_attr,
           species_species_edge_index, species_species_edge_attr, params):
    p = params
    vis = (~traits_nanmask).astype(jnp.float32)
    sp_in = jnp.concatenate(
        [species_x_mean, species_x_std, vis, species_x_gen, species_x_phylo],
        axis=1)
    sp_in = jax.nn.relu(sp_in @ p['W_lin'] + p['b_lin'])

    sp_src, sp_dst, sp_attr, sp_epad = _pad_edges(
        spatial_spatial_edge_index, spatial_spatial_edge_attr, _N_SPATIAL)
    space_in = jnp.concatenate([spatial_x, spatial_global_data], axis=1)
    h, sp_aux = _gat_layer_sc(space_in, space_in, sp_src, sp_dst, sp_attr,
                              sp_epad, p['space_g1'], _N_SPATIAL, _N_SPATIAL,
                              4, 5, False, True)
    h = jax.nn.relu(h)
    space_emb, _ = _gat_layer_sc(h, h, sp_src, sp_dst, sp_attr, sp_epad,
                                 p['space_g2'], _N_SPATIAL, _N_SPATIAL,
                                 4, 2, False, True, aux_in=sp_aux)
    space_emb = jax.nn.relu(space_emb)

    b_src, b_dst, b_attr, b_epad = _pad_edges(
        spatial_species_edge_index, spatial_species_edge_attr, _N_SPECIES)
    species_part = jnp.concatenate([species_x_gen, species_x_phylo], axis=1)
    s2s, _ = _gat_layer_sc(space_emb, species_part, b_src, b_dst, b_attr,
                           b_epad, p['bip'], _N_SPATIAL, _N_SPECIES,
                           1, 5, True, False)
    s2s = jax.nn.relu(s2s)

    ss_src, ss_dst, ss_attr, ss_epad = _pad_edges(
        species_species_edge_index, species_species_edge_attr, _N_SPECIES)
    sp_in = jnp.concatenate([s2s, sp_in], axis=1)
    h, ss_aux = _gat_layer_sc(sp_in, sp_in, ss_src, ss_dst, ss_attr, ss_epad,
                              p['spec_g1'], _N_SPECIES, _N_SPECIES,
                              4, 5, False, True)
    h = jax.nn.relu(h)
    emb, _ = _gat_layer_sc(h, h, ss_src, ss_dst, ss_attr, ss_epad,
                           p['spec_g2'], _N_SPECIES, _N_SPECIES,
                           4, 5, False, True, aux_in=ss_aux)

    out = _fc(emb, p['W_fc'], p['b_fc'])
    pred_mean = out[:, :_OUT]
    pred_std = jax.nn.softplus(out[:, _OUT:]) + _EPS
    return (pred_mean, pred_std)
